# baseline (speedup 1.0000x reference)
import functools
import math

import jax
import jax.numpy as jnp
from jax import lax
from jax.experimental import pallas as pl
from jax.experimental.pallas import tpu as pltpu
from jax.experimental.pallas import tpu_sc as plsc

D_MODEL = 64
SCALE = math.sqrt(D_MODEL)

_NC = 2
_NS = 16
_NW = _NC * _NS

_V = 1000000
_NT_FULL = _V // 128
_T_PER_W = (-(-_NT_FULL // _NW) + 1) // 2 * 2


@jax.jit
def _sc_transpose_table(table_t, tail):
    mesh = plsc.VectorSubcoreMesh(core_axis_name="c", subcore_axis_name="s")

    @functools.partial(
        pl.kernel,
        mesh=mesh,
        out_type=jax.ShapeDtypeStruct((_V // 2, 128), jnp.float32),
        scratch_types=[
            pltpu.VMEM((64, 128), jnp.float32),
            pltpu.VMEM((64, 128), jnp.float32),
            pltpu.VMEM((64, 128), jnp.float32),
            pltpu.VMEM((64, 128), jnp.float32),
            pltpu.SemaphoreType.DMA,
            pltpu.SemaphoreType.DMA,
            pltpu.SemaphoreType.DMA,
            pltpu.SemaphoreType.DMA,
        ],
        compiler_params=pltpu.CompilerParams(use_tc_tiling_on_sc=True, needs_layout_passes=False),
    )
    def k1(tt_hbm, tail_hbm, z_hbm, in0, in1, ou0, ou1, gi0, gi1, go0, go1):
        inb, oub = (in0, in1), (ou0, ou1)
        sem_i, sem_o = (gi0, gi1), (go0, go1)
        wid = lax.axis_index("s") * _NC + lax.axis_index("c")
        t_base = wid * _T_PER_W

        def tile_of(k):
            return jnp.minimum(t_base + k, _NT_FULL - 1)

        def start_read(k, b):
            t = tile_of(k)
            return pltpu.async_copy(
                tt_hbm.at[:, pl.ds(t * 128, 128)], inb[b], sem_i[b])

        def wait_read(b):
            pltpu.make_async_copy(
                tt_hbm.at[:, pl.ds(0, 128)], inb[b], sem_i[b]).wait()

        def start_write(k, b):
            t = tile_of(k)
            return pltpu.async_copy(
                oub[b], z_hbm.at[pl.ds(t * 64, 64), :], sem_o[b])

        def wait_write(b):
            pltpu.make_async_copy(
                oub[b], z_hbm.at[pl.ds(0, 64), :], sem_o[b]).wait()

        rows = [jax.lax.iota(jnp.int32, 16) + 16 * g for g in range(4)]

        def transpose(b):
            def zrow(r, c2):
                for q in range(8):
                    col = jnp.full((16,), 2 * r + (q // 4), jnp.int32)
                    v = plsc.load_gather(inb[b], [rows[q % 4], col])
                    oub[b][r, pl.ds(q * 16, 16)] = v
                return c2
            lax.fori_loop(0, 64, zrow, 0, unroll=2)

        start_read(0, 0)
        start_read(1, 1)
        wait_read(0)
        transpose(0)
        start_write(0, 0)
        start_read(2, 0)
        wait_read(1)
        transpose(1)
        start_write(1, 1)

        def steady(p, carry):
            for t in range(2):
                k = 2 + p * 2 + t
                b = t
                start_read(k + 1, 1 - t)
                wait_read(b)
                wait_write(b)
                transpose(b)
                start_write(k, b)
            return carry

        lax.fori_loop(0, (_T_PER_W - 4) // 2, steady, 0)

        for k in (_T_PER_W - 2, _T_PER_W - 1):
            b = k % 2
            if k + 1 < _T_PER_W:
                start_read(k + 1, (k + 1) % 2)
            wait_read(b)
            wait_write(b)
            transpose(b)
            start_write(k, b)
        for b in range(2):
            wait_write(b)

        @pl.when(wid == _NW - 1)
        def _():
            pltpu.sync_copy(tail_hbm, z_hbm.at[pl.ds(_V // 2 - 32, 32), :])

    return k1(table_t, tail)


@functools.partial(jax.jit, static_argnames=("nrows", "seq"))
def _sc_embed(z1, xf, *, nrows, seq):
    per_w = nrows // _NW * seq
    bpw = nrows // _NW
    mesh = plsc.VectorSubcoreMesh(core_axis_name="c", subcore_axis_name="s")

    @functools.partial(
        pl.kernel,
        mesh=mesh,
        out_type=jax.ShapeDtypeStruct((seq, D_MODEL, nrows), jnp.float32),
        scratch_types=[
            pltpu.VMEM((per_w,), jnp.int32),
            pltpu.VMEM((bpw,), jnp.int32),
            pltpu.VMEM((bpw,), jnp.int32),
            pltpu.VMEM((bpw, D_MODEL), jnp.float32),
            pltpu.VMEM((bpw, D_MODEL), jnp.float32),
            pltpu.VMEM((D_MODEL, bpw), jnp.float32),
            pltpu.VMEM((D_MODEL, bpw), jnp.float32),
            pltpu.SemaphoreType.DMA,
            pltpu.SemaphoreType.DMA,
            pltpu.SemaphoreType.DMA,
            pltpu.SemaphoreType.DMA,
        ],
        compiler_params=pltpu.CompilerParams(
            use_tc_tiling_on_sc=False, needs_layout_passes=False),
    )
    def k2(z_hbm, xf_hbm, out_hbm, xw, ix0, ix1, in0, in1, ou0, ou1,
           gi0, gi1, go0, go1):
        idxb, inb, oub = (ix0, ix1), (in0, in1), (ou0, ou1)
        sem_i, sem_o = (gi0, gi1), (go0, go1)
        wid = lax.axis_index("s") * _NC + lax.axis_index("c")
        b0 = wid * bpw
        pltpu.sync_copy(xf_hbm.at[pl.ds(wid * per_w, per_w)], xw)

        jrow = [jax.lax.iota(jnp.int32, 16) + 16 * g for g in range(8)]
        jpos = [(jax.lax.iota(jnp.int32, 16) + 16 * g) * seq for g in range(8)]

        def start_gather(s, b):
            for g in range(bpw // 16):
                pos = jpos[g] + s
                idxb[b][pl.ds(g * 16, 16)] = plsc.load_gather(xw, [pos])
            return pltpu.async_copy(z_hbm.at[idxb[b]], inb[b], sem_i[b])

        def wait_gather(b):
            pltpu.make_async_copy(z_hbm.at[idxb[b]], inb[b], sem_i[b]).wait()

        def start_write(s, b):
            return pltpu.async_copy(
                oub[b], out_hbm.at[s, :, pl.ds(b0, bpw)], sem_o[b])

        def wait_write(b):
            pltpu.make_async_copy(
                oub[b], out_hbm.at[0, :, pl.ds(b0, bpw)], sem_o[b]).wait()

        def transpose_scale(b):
            def drow(d, c2):
                col = jnp.full((16,), d, jnp.int32)
                for g in range(bpw // 16):
                    v = plsc.load_gather(inb[b], [jrow[g], col])
                    oub[b][d, pl.ds(g * 16, 16)] = v * SCALE
                return c2
            lax.fori_loop(0, D_MODEL, drow, 0, unroll=2)

        start_gather(0, 0)
        start_gather(1, 1)
        wait_gather(0)
        transpose_scale(0)
        start_write(0, 0)
        start_gather(2, 0)
        wait_gather(1)
        transpose_scale(1)
        start_write(1, 1)

        def steady(p, carry):
            for t in range(2):
                s = 2 + p * 2 + t
                b = t
                start_gather(s + 1, 1 - t)
                wait_gather(b)
                wait_write(b)
                transpose_scale(b)
                start_write(s, b)
            return carry

        lax.fori_loop(0, (seq - 4) // 2, steady, 0)

        for s in (seq - 2, seq - 1):
            b = s % 2
            if s + 1 < seq:
                start_gather(s + 1, (s + 1) % 2)
            wait_gather(b)
            wait_write(b)
            transpose_scale(b)
            start_write(s, b)
        for b in range(2):
            wait_write(b)

    return k2(z1, xf)


def kernel(x, table):
    if x.dtype != jnp.int32:
        x = x.astype(jnp.int32)
    nrows, seq = x.shape
    table_t = table.T
    tail = lax.slice(table, (_V - 64, 0), (_V, D_MODEL)).reshape(32, 128)
    z = _sc_transpose_table(table_t, tail)
    z1 = z.reshape(_V, D_MODEL)
    out_t = _sc_embed(z1, x.reshape(nrows * seq), nrows=nrows, seq=seq)
    return jnp.transpose(out_t, (2, 0, 1))


# speedup vs baseline: 2.4168x; 2.4168x over previous
import functools
import math

import jax
import jax.numpy as jnp
from jax import lax
from jax.experimental import pallas as pl
from jax.experimental.pallas import tpu as pltpu
from jax.experimental.pallas import tpu_sc as plsc

D_MODEL = 64
SCALE = math.sqrt(D_MODEL)

_NC = 2
_NS = 16
_NW = _NC * _NS

_V = 1000000
_NT_FULL = _V // 128
_T_PER_W = (-(-_NT_FULL // _NW) + 1) // 2 * 2


@jax.jit
def _sc_transpose_table(table_t, tail):
    mesh = plsc.VectorSubcoreMesh(core_axis_name="c", subcore_axis_name="s")

    @functools.partial(
        pl.kernel,
        mesh=mesh,
        out_type=jax.ShapeDtypeStruct((_V // 2, 128), jnp.float32),
        scratch_types=[
            pltpu.VMEM((64, 128), jnp.float32),
            pltpu.VMEM((64, 128), jnp.float32),
            pltpu.VMEM((64, 128), jnp.float32),
            pltpu.VMEM((64, 128), jnp.float32),
            pltpu.SemaphoreType.DMA,
            pltpu.SemaphoreType.DMA,
            pltpu.SemaphoreType.DMA,
            pltpu.SemaphoreType.DMA,
        ],
        compiler_params=pltpu.CompilerParams(use_tc_tiling_on_sc=True, needs_layout_passes=False),
    )
    def k1(tt_hbm, tail_hbm, z_hbm, in0, in1, ou0, ou1, gi0, gi1, go0, go1):
        inb, oub = (in0, in1), (ou0, ou1)
        sem_i, sem_o = (gi0, gi1), (go0, go1)
        wid = lax.axis_index("s") * _NC + lax.axis_index("c")
        t_base = wid * _T_PER_W

        def tile_of(k):
            return jnp.minimum(t_base + k, _NT_FULL - 1)

        def start_read(k, b):
            t = tile_of(k)
            return pltpu.async_copy(
                tt_hbm.at[:, pl.ds(t * 128, 128)], inb[b], sem_i[b])

        def wait_read(b):
            pltpu.make_async_copy(
                tt_hbm.at[:, pl.ds(0, 128)], inb[b], sem_i[b]).wait()

        def start_write(k, b):
            t = tile_of(k)
            return pltpu.async_copy(
                oub[b], z_hbm.at[pl.ds(t * 64, 64), :], sem_o[b])

        def wait_write(b):
            pltpu.make_async_copy(
                oub[b], z_hbm.at[pl.ds(0, 64), :], sem_o[b]).wait()

        iota = jax.lax.iota(jnp.int32, 16)
        rots = [(iota + i) & 15 for i in range(16)]

        def transpose(b):
            def rblk(rb, c2):
                r0 = rb * 16
                rvecs = [r0 + rots[i] for i in range(16)]
                for half in range(2):
                    icols = [2 * rv + half for rv in rvecs]
                    for cb in range(4):
                        c0 = cb * 16
                        crow = c0 + iota
                        ccol = c0 + half * 64 + iota
                        for i in range(16):
                            v = plsc.load_gather(inb[b], [crow, icols[i]])
                            plsc.store_scatter(oub[b], [rvecs[i], ccol], v)
                return c2
            lax.fori_loop(0, 4, rblk, 0)

        start_read(0, 0)
        start_read(1, 1)
        wait_read(0)
        transpose(0)
        start_write(0, 0)
        start_read(2, 0)
        wait_read(1)
        transpose(1)
        start_write(1, 1)

        def steady(p, carry):
            for t in range(2):
                k = 2 + p * 2 + t
                b = t
                start_read(k + 1, 1 - t)
                wait_read(b)
                wait_write(b)
                transpose(b)
                start_write(k, b)
            return carry

        lax.fori_loop(0, (_T_PER_W - 4) // 2, steady, 0)

        for k in (_T_PER_W - 2, _T_PER_W - 1):
            b = k % 2
            if k + 1 < _T_PER_W:
                start_read(k + 1, (k + 1) % 2)
            wait_read(b)
            wait_write(b)
            transpose(b)
            start_write(k, b)
        for b in range(2):
            wait_write(b)

        @pl.when(wid == _NW - 1)
        def _():
            pltpu.sync_copy(tail_hbm, z_hbm.at[pl.ds(_V // 2 - 32, 32), :])

    return k1(table_t, tail)


@functools.partial(jax.jit, static_argnames=("nrows", "seq"))
def _sc_embed(z1, xf, *, nrows, seq):
    per_w = nrows // _NW * seq
    bpw = nrows // _NW
    mesh = plsc.VectorSubcoreMesh(core_axis_name="c", subcore_axis_name="s")

    @functools.partial(
        pl.kernel,
        mesh=mesh,
        out_type=jax.ShapeDtypeStruct((seq, D_MODEL, nrows), jnp.float32),
        scratch_types=[
            pltpu.VMEM((per_w,), jnp.int32),
            pltpu.VMEM((bpw,), jnp.int32),
            pltpu.VMEM((bpw,), jnp.int32),
            pltpu.VMEM((bpw, D_MODEL), jnp.float32),
            pltpu.VMEM((bpw, D_MODEL), jnp.float32),
            pltpu.VMEM((D_MODEL, bpw), jnp.float32),
            pltpu.VMEM((D_MODEL, bpw), jnp.float32),
            pltpu.SemaphoreType.DMA,
            pltpu.SemaphoreType.DMA,
            pltpu.SemaphoreType.DMA,
            pltpu.SemaphoreType.DMA,
        ],
        compiler_params=pltpu.CompilerParams(
            use_tc_tiling_on_sc=False, needs_layout_passes=False),
    )
    def k2(z_hbm, xf_hbm, out_hbm, xw, ix0, ix1, in0, in1, ou0, ou1,
           gi0, gi1, go0, go1):
        idxb, inb, oub = (ix0, ix1), (in0, in1), (ou0, ou1)
        sem_i, sem_o = (gi0, gi1), (go0, go1)
        wid = lax.axis_index("s") * _NC + lax.axis_index("c")
        b0 = wid * bpw
        pltpu.sync_copy(xf_hbm.at[pl.ds(wid * per_w, per_w)], xw)

        iota = jax.lax.iota(jnp.int32, 16)
        rots = [(iota + i) & 15 for i in range(16)]
        jpos = [(iota + 16 * g) * seq for g in range(8)]

        def start_gather(s, b):
            for g in range(bpw // 16):
                pos = jpos[g] + s
                idxb[b][pl.ds(g * 16, 16)] = plsc.load_gather(xw, [pos])
            return pltpu.async_copy(z_hbm.at[idxb[b]], inb[b], sem_i[b])

        def wait_gather(b):
            pltpu.make_async_copy(z_hbm.at[idxb[b]], inb[b], sem_i[b]).wait()

        def start_write(s, b):
            return pltpu.async_copy(
                oub[b], out_hbm.at[s, :, pl.ds(b0, bpw)], sem_o[b])

        def wait_write(b):
            pltpu.make_async_copy(
                oub[b], out_hbm.at[0, :, pl.ds(b0, bpw)], sem_o[b]).wait()

        def transpose_scale(b):
            def jblk(jb, c2):
                j0 = jb * 16
                jrow = j0 + iota
                for db in range(D_MODEL // 16):
                    d0 = db * 16
                    for i in range(16):
                        dvec = d0 + rots[i]
                        v = plsc.load_gather(inb[b], [jrow, dvec])
                        plsc.store_scatter(oub[b], [dvec, jrow], v * SCALE)
                return c2
            lax.fori_loop(0, bpw // 16, jblk, 0)

        start_gather(0, 0)
        start_gather(1, 1)
        wait_gather(0)
        transpose_scale(0)
        start_write(0, 0)
        start_gather(2, 0)
        wait_gather(1)
        transpose_scale(1)
        start_write(1, 1)

        def steady(p, carry):
            for t in range(2):
                s = 2 + p * 2 + t
                b = t
                start_gather(s + 1, 1 - t)
                wait_gather(b)
                wait_write(b)
                transpose_scale(b)
                start_write(s, b)
            return carry

        lax.fori_loop(0, (seq - 4) // 2, steady, 0)

        for s in (seq - 2, seq - 1):
            b = s % 2
            if s + 1 < seq:
                start_gather(s + 1, (s + 1) % 2)
            wait_gather(b)
            wait_write(b)
            transpose_scale(b)
            start_write(s, b)
        for b in range(2):
            wait_write(b)

    return k2(z1, xf)


def kernel(x, table):
    if x.dtype != jnp.int32:
        x = x.astype(jnp.int32)
    nrows, seq = x.shape
    table_t = table.T
    tail = lax.slice(table, (_V - 64, 0), (_V, D_MODEL)).reshape(32, 128)
    z = _sc_transpose_table(table_t, tail)
    z1 = z.reshape(_V, D_MODEL)
    out_t = _sc_embed(z1, x.reshape(nrows * seq), nrows=nrows, seq=seq)
    return jnp.transpose(out_t, (2, 0, 1))


# speedup vs baseline: 2.8164x; 1.1654x over previous
import functools
import math

import jax
import jax.numpy as jnp
from jax import lax
from jax.experimental import pallas as pl
from jax.experimental.pallas import tpu as pltpu
from jax.experimental.pallas import tpu_sc as plsc

D_MODEL = 64
SCALE = math.sqrt(D_MODEL)

_NC = 2
_NS = 16
_NW = _NC * _NS
_NBUF = 4

_V = 1000000
_NT_FULL = _V // 128
_T_PER_W = (-(-_NT_FULL // _NW) + 1) // 2 * 2


@jax.jit
def _sc_transpose_table(table_t, tail):
    mesh = plsc.VectorSubcoreMesh(core_axis_name="c", subcore_axis_name="s")

    @functools.partial(
        pl.kernel,
        mesh=mesh,
        out_type=jax.ShapeDtypeStruct((_V // 2, 128), jnp.float32),
        scratch_types=[
            pltpu.VMEM((64, 128), jnp.float32),
            pltpu.VMEM((64, 128), jnp.float32),
            pltpu.VMEM((64, 128), jnp.float32),
            pltpu.VMEM((64, 128), jnp.float32),
            pltpu.SemaphoreType.DMA,
            pltpu.SemaphoreType.DMA,
            pltpu.SemaphoreType.DMA,
            pltpu.SemaphoreType.DMA,
        ],
        compiler_params=pltpu.CompilerParams(use_tc_tiling_on_sc=True, needs_layout_passes=False),
    )
    def k1(tt_hbm, tail_hbm, z_hbm, in0, in1, ou0, ou1, gi0, gi1, go0, go1):
        inb, oub = (in0, in1), (ou0, ou1)
        sem_i, sem_o = (gi0, gi1), (go0, go1)
        wid = lax.axis_index("s") * _NC + lax.axis_index("c")
        t_base = wid * _T_PER_W

        def tile_of(k):
            return jnp.minimum(t_base + k, _NT_FULL - 1)

        def start_read(k, b):
            t = tile_of(k)
            return pltpu.async_copy(
                tt_hbm.at[:, pl.ds(t * 128, 128)], inb[b], sem_i[b])

        def wait_read(b):
            pltpu.make_async_copy(
                tt_hbm.at[:, pl.ds(0, 128)], inb[b], sem_i[b]).wait()

        def start_write(k, b):
            t = tile_of(k)
            return pltpu.async_copy(
                oub[b], z_hbm.at[pl.ds(t * 64, 64), :], sem_o[b])

        def wait_write(b):
            pltpu.make_async_copy(
                oub[b], z_hbm.at[pl.ds(0, 64), :], sem_o[b]).wait()

        iota = jax.lax.iota(jnp.int32, 16)
        rots = [(iota + i) & 15 for i in range(16)]

        def transpose(b):
            def rblk(rb, c2):
                r0 = rb * 16
                rvecs = [r0 + rots[i] for i in range(16)]
                for half in range(2):
                    icols = [2 * rv + half for rv in rvecs]
                    for cb in range(4):
                        c0 = cb * 16
                        crow = c0 + iota
                        ccol = c0 + half * 64 + iota
                        for i in range(16):
                            v = plsc.load_gather(inb[b], [crow, icols[i]])
                            plsc.store_scatter(oub[b], [rvecs[i], ccol], v)
                return c2
            lax.fori_loop(0, 4, rblk, 0)

        start_read(0, 0)
        start_read(1, 1)
        wait_read(0)
        transpose(0)
        start_write(0, 0)
        start_read(2, 0)
        wait_read(1)
        transpose(1)
        start_write(1, 1)

        def steady(p, carry):
            for t in range(2):
                k = 2 + p * 2 + t
                b = t
                start_read(k + 1, 1 - t)
                wait_read(b)
                wait_write(b)
                transpose(b)
                start_write(k, b)
            return carry

        lax.fori_loop(0, (_T_PER_W - 4) // 2, steady, 0)

        for k in (_T_PER_W - 2, _T_PER_W - 1):
            b = k % 2
            if k + 1 < _T_PER_W:
                start_read(k + 1, (k + 1) % 2)
            wait_read(b)
            wait_write(b)
            transpose(b)
            start_write(k, b)
        for b in range(2):
            wait_write(b)

        @pl.when(wid == _NW - 1)
        def _():
            pltpu.sync_copy(tail_hbm, z_hbm.at[pl.ds(_V // 2 - 32, 32), :])

    return k1(table_t, tail)


@functools.partial(jax.jit, static_argnames=("nrows", "seq"))
def _sc_embed(table, x, *, nrows, seq):
    rows_per_w = nrows // _NW
    mesh = plsc.VectorSubcoreMesh(core_axis_name="c", subcore_axis_name="s")

    @functools.partial(
        pl.kernel,
        mesh=mesh,
        out_type=jax.ShapeDtypeStruct((nrows, seq, D_MODEL), jnp.float32),
        scratch_types=[
            pltpu.VMEM((rows_per_w, seq), jnp.int32),
        ]
        + [pltpu.VMEM((seq, D_MODEL), jnp.float32) for _ in range(_NBUF)]
        + [pltpu.SemaphoreType.DMA for _ in range(2 * _NBUF)],
        compiler_params=pltpu.CompilerParams(use_tc_tiling_on_sc=False),
    )
    def k(table_hbm, x_hbm, out_hbm, idx_v, *bufs_and_sems):
        bufs = bufs_and_sems[:_NBUF]
        sem_g = bufs_and_sems[_NBUF:2 * _NBUF]
        sem_s = bufs_and_sems[2 * _NBUF:]

        wid = lax.axis_index("s") * _NC + lax.axis_index("c")
        row0 = wid * rows_per_w
        pltpu.sync_copy(x_hbm.at[pl.ds(row0, rows_per_w), :], idx_v)

        def start_gather(g, b):
            return pltpu.async_copy(table_hbm.at[idx_v.at[g]], bufs[b], sem_g[b])

        def wait_gather(g, b):
            pltpu.make_async_copy(
                table_hbm.at[idx_v.at[g]], bufs[b], sem_g[b]).wait()

        def start_scatter(g, b):
            return pltpu.async_copy(bufs[b], out_hbm.at[row0 + g], sem_s[b])

        def wait_scatter(b):
            pltpu.make_async_copy(bufs[b], out_hbm.at[row0], sem_s[b]).wait()

        def scale(b):
            def row_body(i, c2):
                for j in range(D_MODEL // 16):
                    sl = pl.ds(j * 16, 16)
                    bufs[b][i, sl] = bufs[b][i, sl] * SCALE
                return c2
            lax.fori_loop(0, seq, row_body, 0, unroll=4)

        n = rows_per_w
        start_gather(0, 0)
        start_gather(1, 1)
        start_gather(2, 2)
        wait_gather(0, 0)
        scale(0)
        start_scatter(0, 0)
        start_gather(3, 3)
        wait_gather(1, 1)
        scale(1)
        start_scatter(1, 1)

        def steady(p, carry):
            for b in range(_NBUF):
                g = 2 + p * _NBUF + b
                bb = (2 + b) % _NBUF
                bn = b % _NBUF
                wait_scatter(bn)
                start_gather(g + 2, bn)
                wait_gather(g, bb)
                scale(bb)
                start_scatter(g, bb)
            return carry

        lax.fori_loop(0, (n - 4) // _NBUF, steady, 0)

        wait_gather(n - 2, (n - 2) % _NBUF)
        scale((n - 2) % _NBUF)
        start_scatter(n - 2, (n - 2) % _NBUF)
        wait_gather(n - 1, (n - 1) % _NBUF)
        scale((n - 1) % _NBUF)
        start_scatter(n - 1, (n - 1) % _NBUF)
        for b in range(_NBUF):
            wait_scatter(b)

    return k(table, x)


def kernel(x, table):
    if x.dtype != jnp.int32:
        x = x.astype(jnp.int32)
    nrows, seq = x.shape
    table_t = table.T
    tail = lax.slice(table, (_V - 64, 0), (_V, D_MODEL)).reshape(32, 128)
    z = _sc_transpose_table(table_t, tail)
    z1 = z.reshape(_V, D_MODEL)
    return _sc_embed(z1, x, nrows=nrows, seq=seq)
